# Initial kernel scaffold; baseline (speedup 1.0000x reference)
import jax
import jax.numpy as jnp
from jax.experimental import pallas as pl


def kernel(x, w1, b1, bn_gamma, bn_beta, bn_mean, bn_var, wh, bh, ww, bw):
    raise NotImplementedError("write your pallas kernel here")


# speedup vs baseline: 1.0343x; 1.0343x over previous
import functools

import jax
import jax.numpy as jnp
from jax.experimental import pallas as pl
from jax.experimental.pallas import tpu as pltpu

_BN_EPS = 1e-5
_MIB = 1024 * 1024


def _pool_expand_mats(H, W):
    HW = H * W
    s = jnp.arange(HW, dtype=jnp.int32)
    eh = (s // W == jnp.arange(H, dtype=jnp.int32)[:, None]).astype(jnp.float32)
    ew = (s % W == jnp.arange(W, dtype=jnp.int32)[:, None]).astype(jnp.float32)
    p = jnp.concatenate([eh.T / W, ew.T / H], axis=1)
    return p, eh, ew


def _fused_kernel(xf_ref, p_ref, eh_ref, ew_ref,
                  w1_ref, b1_ref, wh_ref, bh_ref, ww_ref, bw_ref,
                  out_ref, *, H):
    xf = xf_ref[...]
    xb = xf.astype(jnp.bfloat16)

    pooled = jnp.dot(xb, p_ref[...], preferred_element_type=jnp.float32)

    y = jnp.dot(w1_ref[...], pooled,
                preferred_element_type=jnp.float32) + b1_ref[...]
    y = jnp.maximum(y, 0.0)

    a_h = jax.nn.sigmoid(
        jnp.dot(wh_ref[...], y[:, :H],
                preferred_element_type=jnp.float32) + bh_ref[...])
    a_w = jax.nn.sigmoid(
        jnp.dot(ww_ref[...], y[:, H:],
                preferred_element_type=jnp.float32) + bw_ref[...])

    gate = (jnp.dot(a_h.astype(jnp.bfloat16), eh_ref[...],
                    preferred_element_type=jnp.float32)
            * jnp.dot(a_w.astype(jnp.bfloat16), ew_ref[...],
                      preferred_element_type=jnp.float32))
    out_ref[...] = (xf * gate).astype(out_ref.dtype)


def kernel(x, w1, b1, bn_gamma, bn_beta, bn_mean, bn_var, wh, bh, ww, bw):
    N, C, H, W = x.shape
    HW = H * W
    T = H + W
    mid = w1.shape[0]

    scale = bn_gamma * jax.lax.rsqrt(bn_var + _BN_EPS)
    w1f = w1 * scale[:, None]
    b1f = ((b1 - bn_mean) * scale + bn_beta).reshape(mid, 1)

    p_mat, eh_mat, ew_mat = _pool_expand_mats(H, W)
    p_bf = p_mat.astype(jnp.bfloat16)
    eh_bf = eh_mat.astype(jnp.bfloat16)
    ew_bf = ew_mat.astype(jnp.bfloat16)

    xf = x.reshape(N, C, HW)

    def rep(shape):
        return pl.BlockSpec(shape, lambda n: (0,) * len(shape))

    flops = N * (2 * C * HW * T + 2 * mid * C * T + 2 * C * mid * T
                 + 2 * C * T * HW + 3 * C * HW)
    cost = pl.CostEstimate(
        flops=int(flops),
        transcendentals=int(N * C * T),
        bytes_accessed=int(4 * 2 * N * C * HW + 2 * (2 * HW * T + 4 * mid * C)))

    out_flat = pl.pallas_call(
        functools.partial(_fused_kernel, H=H),
        out_shape=jax.ShapeDtypeStruct((N, C, HW), x.dtype),
        grid=(N,),
        in_specs=[
            pl.BlockSpec((None, C, HW), lambda n: (n, 0, 0)),
            rep((HW, T)),
            rep((H, HW)),
            rep((W, HW)),
            rep((mid, C)),
            rep((mid, 1)),
            rep((C, mid)),
            rep((C, 1)),
            rep((C, mid)),
            rep((C, 1)),
        ],
        out_specs=pl.BlockSpec((None, C, HW), lambda n: (n, 0, 0)),
        compiler_params=pltpu.CompilerParams(
            dimension_semantics=("parallel",),
            vmem_limit_bytes=48 * _MIB),
        cost_estimate=cost,
    )(xf, p_bf, eh_bf, ew_bf, w1f, b1f, wh,
      bh.reshape(C, 1), ww, bw.reshape(C, 1))
    return out_flat.reshape(N, C, H, W)
